# baseline (speedup 1.0000x reference)
import jax
import jax.numpy as jnp
from jax.experimental import pallas as pl
from jax.experimental.pallas import tpu as pltpu

P = 16
NQ = 100


def _router_kernel(pix_ref, rW_ref, rb_ref, probs_ref, choice_ref, loss_ref,
                   pooled_ref):
    b = pl.program_id(0)
    nb = pl.num_programs(0)
    m = jnp.mean(pix_ref[0], axis=(1, 2))
    pooled_ref[pl.ds(b, 1), :] = m.reshape(1, -1)

    @pl.when(b == nb - 1)
    def _():
        pooled = pooled_ref[:, :]
        rW = rW_ref[:, :]
        logits = jnp.sum(pooled[:, None, :] * rW[None, :, :], axis=2) \
            + rb_ref[0, :][None, :]
        probs = jax.nn.softmax(logits, axis=1)
        probs_ref[:, :] = probs
        choice_ref[0, :] = jnp.argmax(logits, axis=1).astype(jnp.int32)
        e = rW.shape[0]
        usage = jnp.mean(probs, axis=0)
        loss_ref[:, :] = jnp.mean((usage - 1.0 / e) ** 2).reshape(1, 1)


def _expert_kernel(choices_ref, p_ref, w_ref, b_ref, wc_ref, wb_ref,
                   hid_ref, log_ref, box_ref):
    x = p_ref[0].astype(jnp.bfloat16)
    w = w_ref[0]
    h = jnp.dot(x, w, preferred_element_type=jnp.float32)
    h = h + b_ref[0, 0][None, :]
    h = jax.nn.gelu(h)
    hid_ref[0] = h
    q = h[:NQ, :]
    log_ref[0] = jnp.dot(q, wc_ref[0], preferred_element_type=jnp.float32)
    box_ref[0] = jax.nn.sigmoid(
        jnp.dot(q, wb_ref[0], preferred_element_type=jnp.float32))


def kernel(pixel_values, router_W, router_b, expert_patch_W, expert_patch_b,
           expert_cls_W, expert_box_W):
    B, C, H, W = pixel_values.shape
    E, D_in, D = expert_patch_W.shape
    NC = expert_cls_W.shape[2]
    nh, nw = H // P, W // P
    NP = nh * nw

    probs, choices2d, loss2d = pl.pallas_call(
        _router_kernel,
        grid=(B,),
        in_specs=[
            pl.BlockSpec((1, C, H, W), lambda b: (b, 0, 0, 0)),
            pl.BlockSpec((E, C), lambda b: (0, 0)),
            pl.BlockSpec((1, E), lambda b: (0, 0)),
        ],
        out_specs=[
            pl.BlockSpec((B, E), lambda b: (0, 0)),
            pl.BlockSpec((1, B), lambda b: (0, 0)),
            pl.BlockSpec((1, 1), lambda b: (0, 0)),
        ],
        out_shape=[
            jax.ShapeDtypeStruct((B, E), jnp.float32),
            jax.ShapeDtypeStruct((1, B), jnp.int32),
            jax.ShapeDtypeStruct((1, 1), jnp.float32),
        ],
        scratch_shapes=[pltpu.VMEM((B, C), jnp.float32)],
    )(pixel_values, router_W, router_b.reshape(1, E))
    choices = choices2d[0]
    routing_loss = loss2d[0, 0]

    patches = pixel_values.reshape(B, C, nh, P, nw, P)
    patches = patches.transpose(0, 2, 4, 1, 3, 5).reshape(B, NP, C * P * P)
    patch_W16 = expert_patch_W.astype(jnp.bfloat16)

    bp3 = expert_patch_b.reshape(E, 1, D)
    grid_spec = pltpu.PrefetchScalarGridSpec(
        num_scalar_prefetch=1,
        grid=(B,),
        in_specs=[
            pl.BlockSpec((1, NP, D_in), lambda b, ch: (b, 0, 0)),
            pl.BlockSpec((1, D_in, D), lambda b, ch: (ch[b], 0, 0)),
            pl.BlockSpec((1, 1, D), lambda b, ch: (ch[b], 0, 0)),
            pl.BlockSpec((1, D, NC), lambda b, ch: (ch[b], 0, 0)),
            pl.BlockSpec((1, D, 4), lambda b, ch: (ch[b], 0, 0)),
        ],
        out_specs=[
            pl.BlockSpec((1, NP, D), lambda b, ch: (b, 0, 0)),
            pl.BlockSpec((1, NQ, NC), lambda b, ch: (b, 0, 0)),
            pl.BlockSpec((1, NQ, 4), lambda b, ch: (b, 0, 0)),
        ],
    )
    hidden, batch_logits, batch_pred_boxes = pl.pallas_call(
        _expert_kernel,
        grid_spec=grid_spec,
        out_shape=[
            jax.ShapeDtypeStruct((B, NP, D), jnp.float32),
            jax.ShapeDtypeStruct((B, NQ, NC), jnp.float32),
            jax.ShapeDtypeStruct((B, NQ, 4), jnp.float32),
        ],
    )(choices, patches, patch_W16, bp3, expert_cls_W, expert_box_W)

    return (batch_logits, batch_pred_boxes, hidden, probs, choices,
            routing_loss)


# speedup vs baseline: 1.7725x; 1.7725x over previous
import jax
import jax.numpy as jnp
from jax.experimental import pallas as pl
from jax.experimental.pallas import tpu as pltpu

P = 16
NQ = 100


def _router_kernel(pix_ref, rW_ref, rb_ref, probs_ref, choice_ref, loss_ref,
                   pooled_ref):
    b = pl.program_id(0)
    nb = pl.num_programs(0)
    m = jnp.mean(pix_ref[0], axis=(1, 2))
    pooled_ref[pl.ds(b, 1), :] = m.reshape(1, -1)

    @pl.when(b == nb - 1)
    def _():
        pooled = pooled_ref[:, :]
        rW = rW_ref[:, :]
        logits = jnp.sum(pooled[:, None, :] * rW[None, :, :], axis=2) \
            + rb_ref[0, :][None, :]
        probs = jax.nn.softmax(logits, axis=1)
        probs_ref[:, :] = probs
        choice_ref[0, :] = jnp.argmax(logits, axis=1).astype(jnp.int32)
        e = rW.shape[0]
        usage = jnp.mean(probs, axis=0)
        loss_ref[:, :] = jnp.mean((usage - 1.0 / e) ** 2).reshape(1, 1)


def _expert_kernel(choices_ref, p_ref, w_ref, b_ref, wc_ref, wb_ref,
                   hid_ref, log_ref, box_ref):
    pix = p_ref[0]
    cols = []
    for c in range(3):
        T = pix[c].T
        t4 = T.reshape(32, P, 32, P).transpose(2, 0, 1, 3).reshape(1024, 256)
        cols.append(t4)
    x = jnp.concatenate(cols, axis=1).astype(jnp.bfloat16)
    w = w_ref[0]
    h = jnp.dot(x, w, preferred_element_type=jnp.float32)
    h = h + b_ref[0, 0][None, :]
    h = jax.nn.gelu(h)
    hid_ref[0] = h
    q = h[:NQ, :]
    log_ref[0] = jnp.dot(q, wc_ref[0], preferred_element_type=jnp.float32)
    box_ref[0] = jax.nn.sigmoid(
        jnp.dot(q, wb_ref[0], preferred_element_type=jnp.float32))


def kernel(pixel_values, router_W, router_b, expert_patch_W, expert_patch_b,
           expert_cls_W, expert_box_W):
    B, C, H, W = pixel_values.shape
    E, D_in, D = expert_patch_W.shape
    NC = expert_cls_W.shape[2]
    nh, nw = H // P, W // P
    NP = nh * nw

    probs, choices2d, loss2d = pl.pallas_call(
        _router_kernel,
        grid=(B,),
        in_specs=[
            pl.BlockSpec((1, C, H, W), lambda b: (b, 0, 0, 0)),
            pl.BlockSpec((E, C), lambda b: (0, 0)),
            pl.BlockSpec((1, E), lambda b: (0, 0)),
        ],
        out_specs=[
            pl.BlockSpec((B, E), lambda b: (0, 0)),
            pl.BlockSpec((1, B), lambda b: (0, 0)),
            pl.BlockSpec((1, 1), lambda b: (0, 0)),
        ],
        out_shape=[
            jax.ShapeDtypeStruct((B, E), jnp.float32),
            jax.ShapeDtypeStruct((1, B), jnp.int32),
            jax.ShapeDtypeStruct((1, 1), jnp.float32),
        ],
        scratch_shapes=[pltpu.VMEM((B, C), jnp.float32)],
    )(pixel_values, router_W, router_b.reshape(1, E))
    choices = choices2d[0]
    routing_loss = loss2d[0, 0]

    patches = pixel_values.reshape(B, C, nh, P, nw, P)
    patches = patches.transpose(0, 2, 4, 1, 3, 5).reshape(B, NP, C * P * P)
    Wr = expert_patch_W.reshape(E, C, P, P, D).transpose(0, 1, 3, 2, 4)
    patch_W16 = Wr.reshape(E, C * P * P, D).astype(jnp.bfloat16)

    bp3 = expert_patch_b.reshape(E, 1, D)
    grid_spec = pltpu.PrefetchScalarGridSpec(
        num_scalar_prefetch=1,
        grid=(B,),
        in_specs=[
            pl.BlockSpec((1, C, H, W), lambda b, ch: (b, 0, 0, 0)),
            pl.BlockSpec((1, D_in, D), lambda b, ch: (ch[b], 0, 0)),
            pl.BlockSpec((1, 1, D), lambda b, ch: (ch[b], 0, 0)),
            pl.BlockSpec((1, D, NC), lambda b, ch: (ch[b], 0, 0)),
            pl.BlockSpec((1, D, 4), lambda b, ch: (ch[b], 0, 0)),
        ],
        out_specs=[
            pl.BlockSpec((1, NP, D), lambda b, ch: (b, 0, 0)),
            pl.BlockSpec((1, NQ, NC), lambda b, ch: (b, 0, 0)),
            pl.BlockSpec((1, NQ, 4), lambda b, ch: (b, 0, 0)),
        ],
    )
    hidden, batch_logits, batch_pred_boxes = pl.pallas_call(
        _expert_kernel,
        grid_spec=grid_spec,
        out_shape=[
            jax.ShapeDtypeStruct((B, NP, D), jnp.float32),
            jax.ShapeDtypeStruct((B, NQ, NC), jnp.float32),
            jax.ShapeDtypeStruct((B, NQ, 4), jnp.float32),
        ],
    )(choices, pixel_values, patch_W16, bp3, expert_cls_W, expert_box_W)

    return (batch_logits, batch_pred_boxes, hidden, probs, choices,
            routing_loss)


# speedup vs baseline: 1.7973x; 1.0140x over previous
import functools

import jax
import jax.numpy as jnp
from jax import lax
from jax.experimental import pallas as pl
from jax.experimental.pallas import tpu as pltpu
from jax.experimental.pallas import tpu_sc as plsc

P = 16
NQ = 100

_B, _C, _H, _W = 16, 3, 512, 512
_CHUNKS = _B * _C * _H * (_W // 16)
_UNIT = 2048
_NW = 32
_UNITS_PER_W = _CHUNKS // _UNIT // _NW


def _patchify_sc(pix_ref, out_ref, buf, asm):
    wid = lax.axis_index("c") * 16 + lax.axis_index("s")

    def body(t, carry):
        u = wid * 16 + t
        b = u // 32
        a = u % 32
        for c in range(3):
            start = ((b * 3 + c) * 16384 + a * 512) * 16
            pltpu.sync_copy(pix_ref.at[pl.ds(start, 8192)],
                            buf.at[pl.ds(c * 8192, 8192)])

        def shuf(bp, carry2):
            for c in range(3):
                for i in range(16):
                    src = (c * 512 + i * 32 + bp) * 16
                    dst = (bp * 48 + c * 16 + i) * 16
                    asm[pl.ds(dst, 16)] = buf[pl.ds(src, 16)]
            return carry2

        lax.fori_loop(0, 32, shuf, 0)
        d0 = (b * 1024 + a * 32) * 48 * 16
        pltpu.sync_copy(asm, out_ref.at[pl.ds(d0, 24576)])
        return carry

    lax.fori_loop(0, 16, body, 0)


def _router_kernel(pix_ref, rW_ref, rb_ref, probs_ref, choice_ref, loss_ref,
                   pooled_ref):
    b = pl.program_id(0)
    nb = pl.num_programs(0)
    m = jnp.mean(pix_ref[0], axis=(1, 2))
    pooled_ref[pl.ds(b, 1), :] = m.reshape(1, -1)

    @pl.when(b == nb - 1)
    def _():
        pooled = pooled_ref[:, :]
        rW = rW_ref[:, :]
        logits = jnp.sum(pooled[:, None, :] * rW[None, :, :], axis=2) \
            + rb_ref[0, :][None, :]
        probs = jax.nn.softmax(logits, axis=1)
        probs_ref[:, :] = probs
        choice_ref[0, :] = jnp.argmax(logits, axis=1).astype(jnp.int32)
        e = rW.shape[0]
        usage = jnp.mean(probs, axis=0)
        loss_ref[:, :] = jnp.mean((usage - 1.0 / e) ** 2).reshape(1, 1)


def _expert_kernel(choices_ref, p_ref, w_ref, b_ref, wc_ref, wb_ref,
                   hid_ref, log_ref, box_ref):
    x = p_ref[0].astype(jnp.bfloat16)
    w = w_ref[0]
    h = jnp.dot(x, w, preferred_element_type=jnp.float32)
    h = h + b_ref[0, 0][None, :]
    h = jax.nn.gelu(h)
    hid_ref[0] = h
    q = h[:NQ, :]
    log_ref[0] = jnp.dot(q, wc_ref[0], preferred_element_type=jnp.float32)
    box_ref[0] = jax.nn.sigmoid(
        jnp.dot(q, wb_ref[0], preferred_element_type=jnp.float32))


def kernel(pixel_values, router_W, router_b, expert_patch_W, expert_patch_b,
           expert_cls_W, expert_box_W):
    B, C, H, W = pixel_values.shape
    E, D_in, D = expert_patch_W.shape
    NC = expert_cls_W.shape[2]
    nh, nw = H // P, W // P
    NP = nh * nw

    pix_flat = pixel_values.reshape(-1)
    patchify = functools.partial(
        pl.kernel,
        mesh=plsc.VectorSubcoreMesh(core_axis_name="c", subcore_axis_name="s"),
        out_type=jax.ShapeDtypeStruct((_CHUNKS * 16,), jnp.float32),
        scratch_types=[
            pltpu.VMEM((24576,), jnp.float32),
            pltpu.VMEM((24576,), jnp.float32),
        ],
    )(_patchify_sc)
    patches = patchify(pix_flat).reshape(B, NP, C * P * P)

    probs, choices2d, loss2d = pl.pallas_call(
        _router_kernel,
        grid=(B,),
        in_specs=[
            pl.BlockSpec((1, C, H, W), lambda b: (b, 0, 0, 0)),
            pl.BlockSpec((E, C), lambda b: (0, 0)),
            pl.BlockSpec((1, E), lambda b: (0, 0)),
        ],
        out_specs=[
            pl.BlockSpec((B, E), lambda b: (0, 0)),
            pl.BlockSpec((1, B), lambda b: (0, 0)),
            pl.BlockSpec((1, 1), lambda b: (0, 0)),
        ],
        out_shape=[
            jax.ShapeDtypeStruct((B, E), jnp.float32),
            jax.ShapeDtypeStruct((1, B), jnp.int32),
            jax.ShapeDtypeStruct((1, 1), jnp.float32),
        ],
        scratch_shapes=[pltpu.VMEM((B, C), jnp.float32)],
    )(pixel_values, router_W, router_b.reshape(1, E))
    choices = choices2d[0]
    routing_loss = loss2d[0, 0]

    patch_W16 = expert_patch_W.astype(jnp.bfloat16)

    bp3 = expert_patch_b.reshape(E, 1, D)
    grid_spec = pltpu.PrefetchScalarGridSpec(
        num_scalar_prefetch=1,
        grid=(B,),
        in_specs=[
            pl.BlockSpec((1, NP, D_in), lambda b, ch: (b, 0, 0)),
            pl.BlockSpec((1, D_in, D), lambda b, ch: (ch[b], 0, 0)),
            pl.BlockSpec((1, 1, D), lambda b, ch: (ch[b], 0, 0)),
            pl.BlockSpec((1, D, NC), lambda b, ch: (ch[b], 0, 0)),
            pl.BlockSpec((1, D, 4), lambda b, ch: (ch[b], 0, 0)),
        ],
        out_specs=[
            pl.BlockSpec((1, NP, D), lambda b, ch: (b, 0, 0)),
            pl.BlockSpec((1, NQ, NC), lambda b, ch: (b, 0, 0)),
            pl.BlockSpec((1, NQ, 4), lambda b, ch: (b, 0, 0)),
        ],
    )
    hidden, batch_logits, batch_pred_boxes = pl.pallas_call(
        _expert_kernel,
        grid_spec=grid_spec,
        out_shape=[
            jax.ShapeDtypeStruct((B, NP, D), jnp.float32),
            jax.ShapeDtypeStruct((B, NQ, NC), jnp.float32),
            jax.ShapeDtypeStruct((B, NQ, 4), jnp.float32),
        ],
    )(choices, patches, patch_W16, bp3, expert_cls_W, expert_box_W)

    return (batch_logits, batch_pred_boxes, hidden, probs, choices,
            routing_loss)


# speedup vs baseline: 3.4896x; 1.9416x over previous
import functools

import jax
import jax.numpy as jnp
from jax import lax
from jax.experimental import pallas as pl
from jax.experimental.pallas import tpu as pltpu
from jax.experimental.pallas import tpu_sc as plsc

P = 16
NQ = 100

_B, _C, _H, _W = 16, 3, 512, 512
_CHUNKS = _B * _C * _H * (_W // 16)
_UNIT = 2048
_NW = 32
_UNITS_PER_W = _CHUNKS // _UNIT // _NW


def _patchify_sc(pix_ref, out_ref, buf, asm, sem_l, sem_s):
    wid = lax.axis_index("c") * 16 + lax.axis_index("s")

    def issue_loads(t, slot):
        u = wid * 16 + t
        b = u // 32
        a = u % 32
        for c in range(3):
            row0 = (b * 3 + c) * 512 + a * 16
            pltpu.make_async_copy(
                pix_ref.at[pl.ds(row0, 16), :],
                buf.at[slot, pl.ds(c * 16, 16), :], sem_l).start()

    issue_loads(0, 0)

    def body(t, carry):
        slot = lax.rem(t, 2)
        u = wid * 16 + t
        b = u // 32
        a = u % 32
        for c in range(3):
            pltpu.make_async_copy(
                pix_ref.at[pl.ds(0, 16), :],
                buf.at[slot, pl.ds(c * 16, 16), :], sem_l).wait()

        @pl.when(t + 1 < 16)
        def _():
            issue_loads(t + 1, lax.rem(t + 1, 2))

        @pl.when(t >= 2)
        def _():
            pltpu.make_async_copy(
                asm.at[slot], out_ref.at[pl.ds(0, 32), :], sem_s).wait()

        def shuf(bp, carry2):
            for c in range(3):
                for i in range(16):
                    asm[slot, bp, pl.ds((c * 16 + i) * 16, 16)] = \
                        buf[slot, c * 16 + i, pl.ds(bp * 16, 16)]
            return carry2

        lax.fori_loop(0, 32, shuf, 0)
        p0 = b * 1024 + a * 32
        pltpu.make_async_copy(
            asm.at[slot], out_ref.at[pl.ds(p0, 32), :], sem_s).start()
        return carry

    lax.fori_loop(0, 16, body, 0)
    for t in (14, 15):
        pltpu.make_async_copy(
            asm.at[t % 2], out_ref.at[pl.ds(0, 32), :], sem_s).wait()


def _router_kernel(pix_ref, rW_ref, rb_ref, probs_ref, choice_ref, loss_ref,
                   pooled_ref):
    b = pl.program_id(0)
    nb = pl.num_programs(0)
    m = jnp.mean(pix_ref[0], axis=(1, 2))
    pooled_ref[pl.ds(b, 1), :] = m.reshape(1, -1)

    @pl.when(b == nb - 1)
    def _():
        pooled = pooled_ref[:, :]
        rW = rW_ref[:, :]
        logits = jnp.sum(pooled[:, None, :] * rW[None, :, :], axis=2) \
            + rb_ref[0, :][None, :]
        probs = jax.nn.softmax(logits, axis=1)
        probs_ref[:, :] = probs
        choice_ref[0, :] = jnp.argmax(logits, axis=1).astype(jnp.int32)
        e = rW.shape[0]
        usage = jnp.mean(probs, axis=0)
        loss_ref[:, :] = jnp.mean((usage - 1.0 / e) ** 2).reshape(1, 1)


def _expert_kernel(choices_ref, p_ref, w_ref, b_ref, wc_ref, wb_ref,
                   hid_ref, log_ref, box_ref):
    x = p_ref[0].astype(jnp.bfloat16)
    w = w_ref[0]
    h = jnp.dot(x, w, preferred_element_type=jnp.float32)
    h = h + b_ref[0, 0][None, :]
    h = jax.nn.gelu(h)
    hid_ref[0] = h
    q = h[:NQ, :]
    log_ref[0] = jnp.dot(q, wc_ref[0], preferred_element_type=jnp.float32)
    box_ref[0] = jax.nn.sigmoid(
        jnp.dot(q, wb_ref[0], preferred_element_type=jnp.float32))


def kernel(pixel_values, router_W, router_b, expert_patch_W, expert_patch_b,
           expert_cls_W, expert_box_W):
    B, C, H, W = pixel_values.shape
    E, D_in, D = expert_patch_W.shape
    NC = expert_cls_W.shape[2]
    nh, nw = H // P, W // P
    NP = nh * nw

    pix2d = pixel_values.reshape(B * C * H, W)
    patchify = functools.partial(
        pl.kernel,
        mesh=plsc.VectorSubcoreMesh(core_axis_name="c", subcore_axis_name="s"),
        out_type=jax.ShapeDtypeStruct((B * NP, C * P * P), jnp.float32),
        scratch_types=[
            pltpu.VMEM((2, 48, 512), jnp.float32),
            pltpu.VMEM((2, 32, 768), jnp.float32),
            pltpu.SemaphoreType.DMA,
            pltpu.SemaphoreType.DMA,
        ],
    )(_patchify_sc)
    patches = patchify(pix2d).reshape(B, NP, C * P * P)

    probs, choices2d, loss2d = pl.pallas_call(
        _router_kernel,
        grid=(B,),
        in_specs=[
            pl.BlockSpec((1, C, H, W), lambda b: (b, 0, 0, 0)),
            pl.BlockSpec((E, C), lambda b: (0, 0)),
            pl.BlockSpec((1, E), lambda b: (0, 0)),
        ],
        out_specs=[
            pl.BlockSpec((B, E), lambda b: (0, 0)),
            pl.BlockSpec((1, B), lambda b: (0, 0)),
            pl.BlockSpec((1, 1), lambda b: (0, 0)),
        ],
        out_shape=[
            jax.ShapeDtypeStruct((B, E), jnp.float32),
            jax.ShapeDtypeStruct((1, B), jnp.int32),
            jax.ShapeDtypeStruct((1, 1), jnp.float32),
        ],
        scratch_shapes=[pltpu.VMEM((B, C), jnp.float32)],
    )(pixel_values, router_W, router_b.reshape(1, E))
    choices = choices2d[0]
    routing_loss = loss2d[0, 0]

    patch_W16 = expert_patch_W.astype(jnp.bfloat16)

    bp3 = expert_patch_b.reshape(E, 1, D)
    grid_spec = pltpu.PrefetchScalarGridSpec(
        num_scalar_prefetch=1,
        grid=(B,),
        in_specs=[
            pl.BlockSpec((1, NP, D_in), lambda b, ch: (b, 0, 0)),
            pl.BlockSpec((1, D_in, D), lambda b, ch: (ch[b], 0, 0)),
            pl.BlockSpec((1, 1, D), lambda b, ch: (ch[b], 0, 0)),
            pl.BlockSpec((1, D, NC), lambda b, ch: (ch[b], 0, 0)),
            pl.BlockSpec((1, D, 4), lambda b, ch: (ch[b], 0, 0)),
        ],
        out_specs=[
            pl.BlockSpec((1, NP, D), lambda b, ch: (b, 0, 0)),
            pl.BlockSpec((1, NQ, NC), lambda b, ch: (b, 0, 0)),
            pl.BlockSpec((1, NQ, 4), lambda b, ch: (b, 0, 0)),
        ],
    )
    hidden, batch_logits, batch_pred_boxes = pl.pallas_call(
        _expert_kernel,
        grid_spec=grid_spec,
        out_shape=[
            jax.ShapeDtypeStruct((B, NP, D), jnp.float32),
            jax.ShapeDtypeStruct((B, NQ, NC), jnp.float32),
            jax.ShapeDtypeStruct((B, NQ, 4), jnp.float32),
        ],
    )(choices, patches, patch_W16, bp3, expert_cls_W, expert_box_W)

    return (batch_logits, batch_pred_boxes, hidden, probs, choices,
            routing_loss)
